# baseline (speedup 1.0000x reference)
import jax
import jax.numpy as jnp
from jax import lax
from jax.experimental import pallas as pl
from jax.experimental.pallas import tpu as pltpu
from jax.experimental.pallas import tpu_sc as plsc

_VOCAB = 100000
_D = 128
_HID = 512
_B = 4096
_SEQ = 200

_NC = 2
_NS = 16
_NW = _NC * _NS
_BPW = _B // _NW
_CHUNK = 128
_NCHUNK = _BPW * _SEQ // _CHUNK
_NBUF = 4
_K = 2
_SEC = 40
_NSEC = _NCHUNK // _SEC
_ACC_ROWS = _NS * _BPW
_ALU_SLOTS = (0,)
_NV = _D // 16


def _pool_body(idx_hbm, scat_hbm, table_hbm, out_hbm,
               idx_v, scat_v, buf0, buf1, buf2, buf3, acc_loc, acc,
               g0, g1, g2, g3, s0, s1, s2, s3, isem):
    bufs = (buf0, buf1, buf2, buf3)
    gsems = (g0, g1, g2, g3)
    ssems = (s0, s1, s2, s3)
    c = lax.axis_index("c")
    s = lax.axis_index("s")
    wid = c * _NS + s

    pltpu.sync_copy(idx_hbm.at[wid, 0], idx_v.at[0])
    pltpu.sync_copy(scat_hbm.at[s, 0], scat_v.at[0])

    zero16 = jnp.zeros((16,), jnp.float32)

    @pl.loop(0, _BPW)
    def _(r):
        @pl.loop(0, _D, step=16)
        def _(k):
            acc_loc[r, pl.ds(k, 16)] = zero16

    pltpu.sync_copy(acc_loc, acc.at[pl.ds(s * _BPW, _BPW)])

    def idx_row(jj):
        return idx_v.at[(jj // _SEC) % 2, jj % _SEC]

    def scat_row(jj):
        return scat_v.at[(jj // _SEC) % 2, jj % _SEC]

    for b in range(_NBUF - _K):
        pltpu.async_copy(table_hbm.at[idx_row(b)], bufs[b], gsems[b])

    def alu_reduce(jj, b):
        p0 = jj * _CHUNK
        r0 = p0 // _SEQ
        bnd = jnp.minimum(_SEQ * (r0 + 1) - p0, _CHUNK)
        r1 = (p0 + _CHUNK - 1) // _SEQ

        def body(r, regs):
            return tuple(
                regs[m] + bufs[b][r, pl.ds(16 * m, 16)]
                for m in range(_NV))

        def flush(row, regs):
            for m in range(_NV):
                acc_loc[row, pl.ds(16 * m, 16)] = (
                    acc_loc[row, pl.ds(16 * m, 16)] + regs[m])

        init = (zero16,) * _NV
        flush(r0, lax.fori_loop(0, bnd, body, init))
        flush(r1, lax.fori_loop(bnd, _CHUNK, body, init))

    def slot(jj, b):
        b2 = (b + _NBUF - _K) % _NBUF
        if b == 0:
            @pl.when((jj % _SEC == 0) & (jj > 0))
            def _():
                pltpu.make_async_copy(
                    idx_hbm.at[wid, 0], idx_v.at[0], isem).wait()
                pltpu.make_async_copy(
                    scat_hbm.at[s, 0], scat_v.at[0], isem).wait()
        if b == _K:
            e = jj // _SEC
            @pl.when((jj % _SEC == _K) & (jj < _NCHUNK - _SEC))
            def _():
                pltpu.async_copy(
                    idx_hbm.at[wid, e + 1], idx_v.at[(e + 1) % 2], isem)
                pltpu.async_copy(
                    scat_hbm.at[s, e + 1], scat_v.at[(e + 1) % 2], isem)
        pltpu.make_async_copy(
            table_hbm.at[idx_row(jj)], bufs[b], gsems[b]).wait()
        nxt = jj + _NBUF - _K
        if b not in _ALU_SLOTS:
            pltpu.async_copy(bufs[b], acc.at[scat_row(jj)], ssems[b],
                             add=True)

        @pl.when(nxt < _NCHUNK)
        def _():
            if b2 not in _ALU_SLOTS:
                @pl.when(jj >= _K)
                def _():
                    pltpu.make_async_copy(
                        bufs[b2], acc.at[scat_row(jj - _K)],
                        ssems[b2]).wait()
            pltpu.async_copy(
                table_hbm.at[idx_row(nxt)], bufs[b2], gsems[b2])
        if b in _ALU_SLOTS:
            alu_reduce(jj, b)

    @pl.loop(0, _NCHUNK, step=_NBUF)
    def _(j):
        for b in range(_NBUF):
            slot(j + b, b)

    for jj in range(_NCHUNK - _NBUF, _NCHUNK):
        bd = jj % _NBUF
        if bd not in _ALU_SLOTS and (jj + _NBUF - _K) >= _NCHUNK:
            pltpu.make_async_copy(bufs[bd], acc.at[scat_row(jj)],
                                  ssems[bd]).wait()
    pltpu.sync_copy(acc.at[pl.ds(s * _BPW, _BPW)],
                    out_hbm.at[pl.ds(wid * _BPW, _BPW)])
    pltpu.sync_copy(acc_loc, out_hbm.at[pl.ds(_B + wid * _BPW, _BPW)])


@jax.jit
def _sc_pool(idx3, scat_map, table):
    mesh = plsc.VectorSubcoreMesh(core_axis_name="c", subcore_axis_name="s")
    f = pl.kernel(
        _pool_body,
        out_type=jax.ShapeDtypeStruct((2 * _B, _D), jnp.float32),
        mesh=mesh,
        scratch_types=[
            pltpu.VMEM((2, _SEC, _CHUNK), jnp.int32),
            pltpu.VMEM((2, _SEC, _CHUNK), jnp.int32),
        ] + [pltpu.VMEM((_CHUNK, _D), jnp.float32)] * _NBUF
          + [pltpu.VMEM((_BPW, _D), jnp.float32)]
          + [pltpu.VMEM_SHARED((_ACC_ROWS, _D), jnp.float32)]
          + [pltpu.SemaphoreType.DMA] * (2 * _NBUF + 1),
    )
    return f(idx3, scat_map, table)


def _mlp_body(x_ref, w1_ref, b1_ref, w2_ref, b2_ref, o_ref):
    x = (x_ref[0] + x_ref[1]) * jnp.float32(1.0 / _SEQ)
    h = jnp.dot(x, w1_ref[...], preferred_element_type=jnp.float32)
    h = jnp.maximum(h + b1_ref[...], 0.0)
    z = jnp.dot(h, w2_ref[...], preferred_element_type=jnp.float32)
    z = z + b2_ref[...]
    m = jnp.max(z, axis=1, keepdims=True)
    e = jnp.exp(z - m)
    lse = jnp.log(jnp.sum(e, axis=1, keepdims=True)) + m
    o_ref[...] = z - lse


@jax.jit
def _tc_mlp(pooled, W1, b1, W2pad, b2pad):
    return pl.pallas_call(
        _mlp_body,
        out_shape=jax.ShapeDtypeStruct((_B, _D), jnp.float32),
    )(pooled, W1, b1, W2pad, b2pad)


@jax.jit
def kernel(sentence_indices, table, W1, b1, W2, b2):
    idx4 = sentence_indices.astype(jnp.int32).reshape(
        _NW, _NSEC, _SEC, _CHUNK)
    rel = (jnp.arange(_BPW * _SEQ, dtype=jnp.int32) // _SEQ).reshape(
        _NSEC, _SEC, _CHUNK)
    scat_map = rel[None] + (jnp.arange(_NS, dtype=jnp.int32) * _BPW)[
        :, None, None, None]
    pooled = _sc_pool(idx4, scat_map, table).reshape(2, _B, _D)

    W2pad = jnp.zeros((_HID, _D), jnp.float32).at[:, :2].set(W2)
    b2pad = jnp.full((1, _D), -1e30, jnp.float32).at[0, :2].set(b2)
    out = _tc_mlp(pooled, W1, b1.reshape(1, _HID), W2pad, b2pad)
    return out[:, :2]


# speedup vs baseline: 1.0738x; 1.0738x over previous
import jax
import jax.numpy as jnp
from jax import lax
from jax.experimental import pallas as pl
from jax.experimental.pallas import tpu as pltpu
from jax.experimental.pallas import tpu_sc as plsc

_VOCAB = 100000
_D = 128
_HID = 512
_B = 4096
_SEQ = 200

_NC = 2
_NS = 16
_NW = _NC * _NS
_BPW = _B // _NW
_CHUNK = 128
_NCHUNK = _BPW * _SEQ // _CHUNK
_NBUF = 4
_K = 2
_SEC = 40
_NSEC = _NCHUNK // _SEC
_ACC_ROWS = _NS * _BPW
_ALU_SLOTS = (0, 2)
_NV = _D // 16


def _pool_body(idx_hbm, scat_hbm, table_hbm, out_hbm,
               idx_v, scat_v, buf0, buf1, buf2, buf3, acc_loc, acc,
               g0, g1, g2, g3, s0, s1, s2, s3, isem):
    bufs = (buf0, buf1, buf2, buf3)
    gsems = (g0, g1, g2, g3)
    ssems = (s0, s1, s2, s3)
    c = lax.axis_index("c")
    s = lax.axis_index("s")
    wid = c * _NS + s

    pltpu.sync_copy(idx_hbm.at[wid, 0], idx_v.at[0])
    pltpu.sync_copy(scat_hbm.at[s, 0], scat_v.at[0])

    zero16 = jnp.zeros((16,), jnp.float32)

    @pl.loop(0, _BPW)
    def _(r):
        @pl.loop(0, _D, step=16)
        def _(k):
            acc_loc[r, pl.ds(k, 16)] = zero16

    pltpu.sync_copy(acc_loc, acc.at[pl.ds(s * _BPW, _BPW)])

    def idx_row(jj):
        return idx_v.at[(jj // _SEC) % 2, jj % _SEC]

    def scat_row(jj):
        return scat_v.at[(jj // _SEC) % 2, jj % _SEC]

    for b in range(_NBUF - _K):
        pltpu.async_copy(table_hbm.at[idx_row(b)], bufs[b], gsems[b])

    def alu_reduce(jj, b):
        p0 = jj * _CHUNK
        r0 = p0 // _SEQ
        bnd = jnp.minimum(_SEQ * (r0 + 1) - p0, _CHUNK)
        r1 = (p0 + _CHUNK - 1) // _SEQ

        def body(r, regs):
            return tuple(
                regs[m] + bufs[b][r, pl.ds(16 * m, 16)]
                for m in range(_NV))

        def flush(row, regs):
            for m in range(_NV):
                acc_loc[row, pl.ds(16 * m, 16)] = (
                    acc_loc[row, pl.ds(16 * m, 16)] + regs[m])

        def pbody(r, regs):
            return body(r, regs)

        init = (zero16,) * _NV
        flush(r0, plsc.parallel_loop(0, bnd, carry=init)(pbody))
        flush(r1, plsc.parallel_loop(bnd, _CHUNK, carry=init)(pbody))

    def slot(jj, b):
        b2 = (b + _NBUF - _K) % _NBUF
        if b == 0:
            @pl.when((jj % _SEC == 0) & (jj > 0))
            def _():
                pltpu.make_async_copy(
                    idx_hbm.at[wid, 0], idx_v.at[0], isem).wait()
                pltpu.make_async_copy(
                    scat_hbm.at[s, 0], scat_v.at[0], isem).wait()
        if b == _K:
            e = jj // _SEC
            @pl.when((jj % _SEC == _K) & (jj < _NCHUNK - _SEC))
            def _():
                pltpu.async_copy(
                    idx_hbm.at[wid, e + 1], idx_v.at[(e + 1) % 2], isem)
                pltpu.async_copy(
                    scat_hbm.at[s, e + 1], scat_v.at[(e + 1) % 2], isem)
        pltpu.make_async_copy(
            table_hbm.at[idx_row(jj)], bufs[b], gsems[b]).wait()
        nxt = jj + _NBUF - _K
        if b in _ALU_SLOTS:
            @pl.when(nxt < _NCHUNK)
            def _():
                pltpu.async_copy(
                    table_hbm.at[idx_row(nxt)], bufs[b2], gsems[b2])
            alu_reduce(jj, b)
        else:
            pltpu.async_copy(bufs[b], acc.at[scat_row(jj)], ssems[b],
                             add=True)

            @pl.when(nxt < _NCHUNK)
            def _():
                @pl.when(jj >= _K)
                def _():
                    pltpu.make_async_copy(
                        bufs[b2], acc.at[scat_row(jj - _K)],
                        ssems[b2]).wait()
                pltpu.async_copy(
                    table_hbm.at[idx_row(nxt)], bufs[b2], gsems[b2])

    @pl.loop(0, _NCHUNK, step=_NBUF)
    def _(j):
        for b in range(_NBUF):
            slot(j + b, b)

    for jj in range(_NCHUNK - _NBUF, _NCHUNK):
        bd = jj % _NBUF
        if bd not in _ALU_SLOTS and (jj + _NBUF - _K) >= _NCHUNK:
            pltpu.make_async_copy(bufs[bd], acc.at[scat_row(jj)],
                                  ssems[bd]).wait()
    pltpu.sync_copy(acc.at[pl.ds(s * _BPW, _BPW)],
                    out_hbm.at[pl.ds(wid * _BPW, _BPW)])
    pltpu.sync_copy(acc_loc, out_hbm.at[pl.ds(_B + wid * _BPW, _BPW)])


@jax.jit
def _sc_pool(idx3, scat_map, table):
    mesh = plsc.VectorSubcoreMesh(core_axis_name="c", subcore_axis_name="s")
    f = pl.kernel(
        _pool_body,
        out_type=jax.ShapeDtypeStruct((2 * _B, _D), jnp.float32),
        mesh=mesh,
        scratch_types=[
            pltpu.VMEM((2, _SEC, _CHUNK), jnp.int32),
            pltpu.VMEM((2, _SEC, _CHUNK), jnp.int32),
        ] + [pltpu.VMEM((_CHUNK, _D), jnp.float32)] * _NBUF
          + [pltpu.VMEM((_BPW, _D), jnp.float32)]
          + [pltpu.VMEM_SHARED((_ACC_ROWS, _D), jnp.float32)]
          + [pltpu.SemaphoreType.DMA] * (2 * _NBUF + 1),
    )
    return f(idx3, scat_map, table)


def _mlp_body(x_ref, w1_ref, b1_ref, w2_ref, b2_ref, o_ref):
    x = (x_ref[0] + x_ref[1]) * jnp.float32(1.0 / _SEQ)
    h = jnp.dot(x, w1_ref[...], preferred_element_type=jnp.float32)
    h = jnp.maximum(h + b1_ref[...], 0.0)
    z = jnp.dot(h, w2_ref[...], preferred_element_type=jnp.float32)
    z = z + b2_ref[...]
    m = jnp.max(z, axis=1, keepdims=True)
    e = jnp.exp(z - m)
    lse = jnp.log(jnp.sum(e, axis=1, keepdims=True)) + m
    o_ref[...] = z - lse


@jax.jit
def _tc_mlp(pooled, W1, b1, W2pad, b2pad):
    return pl.pallas_call(
        _mlp_body,
        out_shape=jax.ShapeDtypeStruct((_B, _D), jnp.float32),
    )(pooled, W1, b1, W2pad, b2pad)


@jax.jit
def kernel(sentence_indices, table, W1, b1, W2, b2):
    idx4 = sentence_indices.astype(jnp.int32).reshape(
        _NW, _NSEC, _SEC, _CHUNK)
    rel = (jnp.arange(_BPW * _SEQ, dtype=jnp.int32) // _SEQ).reshape(
        _NSEC, _SEC, _CHUNK)
    scat_map = rel[None] + (jnp.arange(_NS, dtype=jnp.int32) * _BPW)[
        :, None, None, None]
    pooled = _sc_pool(idx4, scat_map, table).reshape(2, _B, _D)

    W2pad = jnp.zeros((_HID, _D), jnp.float32).at[:, :2].set(W2)
    b2pad = jnp.full((1, _D), -1e30, jnp.float32).at[0, :2].set(b2)
    out = _tc_mlp(pooled, W1, b1.reshape(1, _HID), W2pad, b2pad)
    return out[:, :2]
